# Initial kernel scaffold; baseline (speedup 1.0000x reference)
import jax
import jax.numpy as jnp
from jax.experimental import pallas as pl


def kernel(x, edge_index, edge_weight, W, b, skip_weight):
    raise NotImplementedError("write your pallas kernel here")


# speedup vs baseline: 6.4445x; 6.4445x over previous
import functools

import jax
import jax.numpy as jnp
from jax import lax
from jax.experimental import pallas as pl
from jax.experimental.pallas import tpu as pltpu
from jax.experimental.pallas import tpu_sc as plsc

N = 10000
E = 320000
D = 128

NC = 2
NS = 16
NW = NC * NS

CH = 80
EPW = E // NW
NCHUNK = EPW // CH
RPT = 632
N_PAD = NS * RPT

_SELU_ALPHA = 1.6732632423543772
_SELU_SCALE = 1.0507009873554805


def _mm_body(x_ref, wt_ref, b_ref, o_ref):
    o_ref[...] = (
        jnp.dot(x_ref[...], wt_ref[...], preferred_element_type=jnp.float32)
        + b_ref[...]
    )


def _matmul(x, wt, b2):
    blk = 2000
    grid = (N // blk,)
    return pl.pallas_call(
        _mm_body,
        grid=grid,
        in_specs=[
            pl.BlockSpec((blk, D), lambda i: (i, 0)),
            pl.BlockSpec((D, D), lambda i: (0, 0)),
            pl.BlockSpec((1, D), lambda i: (0, 0)),
        ],
        out_specs=pl.BlockSpec((blk, D), lambda i: (i, 0)),
        out_shape=jax.ShapeDtypeStruct((N, D), jnp.float32),
    )(x, wt, b2)


def _sc_body(t_hbm, src_hbm, dst_hbm, w_hbm, z_hbm, out_hbm,
             acc, srcbuf, dstbuf, wbuf, rows, sem):
    cid = lax.axis_index("c")
    sid = lax.axis_index("s")
    wid = cid * NS + sid

    pltpu.sync_copy(src_hbm.at[pl.ds(wid * EPW, EPW)], srcbuf)
    pltpu.sync_copy(dst_hbm.at[wid], dstbuf)
    pltpu.sync_copy(w_hbm.at[pl.ds(wid * EPW, EPW)], wbuf)

    pltpu.sync_copy(z_hbm, acc.at[pl.ds(sid * RPT, RPT)])
    plsc.subcore_barrier()

    def _chunk(c, carry):
        pltpu.async_copy(
            t_hbm.at[srcbuf.at[pl.ds(c * CH, CH)]], rows, sem
        ).wait()

        def _grp(g, carry2):
            wv16 = wbuf[pl.ds(c * CH + g * 16, 16)]
            for e16 in range(16):
                wsp = wv16.at[jnp.full((16,), e16, jnp.int32)].get(
                    mode="promise_in_bounds"
                )
                r = g * 16 + e16
                for j in range(D // 16):
                    sl = pl.ds(j * 16, 16)
                    rows[r, sl] = rows[r, sl] * wsp
            return carry2

        lax.fori_loop(0, CH // 16, _grp, 0)
        pltpu.sync_copy(rows, acc.at[dstbuf.at[c]], add=True)
        return carry

    lax.fori_loop(0, NCHUNK, _chunk, 0)

    plsc.subcore_barrier()
    pltpu.sync_copy(
        acc.at[pl.ds(sid * RPT, RPT)],
        out_hbm.at[pl.ds(cid * N_PAD + sid * RPT, RPT)],
    )


def _scatter(transformed, src, dst3d, w, zrows):
    mesh = plsc.VectorSubcoreMesh(core_axis_name="c", subcore_axis_name="s")
    return pl.kernel(
        _sc_body,
        out_type=jax.ShapeDtypeStruct((NC * N_PAD, D), jnp.float32),
        mesh=mesh,
        scratch_types=[
            pltpu.VMEM_SHARED((N_PAD, D), jnp.float32),
            pltpu.VMEM((EPW,), jnp.int32),
            pltpu.VMEM((NCHUNK, CH), jnp.int32),
            pltpu.VMEM((EPW,), jnp.float32),
            pltpu.VMEM((CH, D), jnp.float32),
            pltpu.SemaphoreType.DMA,
        ],
    )(transformed, src, dst3d, w, zrows)


def _fin_body(t_ref, p0_ref, p1_ref, skip_ref, o_ref):
    z = skip_ref[...] * t_ref[...] + p0_ref[...] + p1_ref[...]
    o_ref[...] = _SELU_SCALE * jnp.where(
        z > 0, z, _SELU_ALPHA * (jnp.exp(z) - 1.0)
    )


def _finish(transformed, p0, p1, skip2):
    blk = 2000
    grid = (N // blk,)
    bs = pl.BlockSpec((blk, D), lambda i: (i, 0))
    return pl.pallas_call(
        _fin_body,
        grid=grid,
        in_specs=[bs, bs, bs, pl.BlockSpec((1, D), lambda i: (0, 0))],
        out_specs=bs,
        out_shape=jax.ShapeDtypeStruct((N, D), jnp.float32),
    )(transformed, p0, p1, skip2)


@jax.jit
def kernel(x, edge_index, edge_weight, W, b, skip_weight):
    transformed = _matmul(x, W.T, b.reshape(1, D))
    src = edge_index[1].astype(jnp.int32)
    dst3d = edge_index[0].astype(jnp.int32).reshape(NW, NCHUNK, CH)
    zrows = jnp.zeros((RPT, D), jnp.float32)
    partials = _scatter(transformed, src, dst3d, edge_weight, zrows)
    return _finish(
        transformed,
        partials[:N],
        partials[N_PAD:N_PAD + N],
        skip_weight.reshape(1, D),
    )
